# baseline (device time: 260230 ns/iter reference)
import jax
import jax.numpy as jnp
from jax import lax
from jax.experimental import pallas as pl
from jax.experimental.pallas import tpu as pltpu

N_DEV = 4
KT = 512
COMM = True


def kernel(x, w_mat):
    M, K = x.shape
    _, N = w_mat.shape
    NB = N // N_DEV
    NK = K // KT

    def body(pos_ref, x_ref, w_ref, out_ref, acc_ref, sendb_ref, stage_ref,
             send_sems, recv_sems, copy_sem):
        j = pl.program_id(0)
        k = pl.program_id(1)
        pos = pos_ref[0]
        dest = lax.rem(pos + 1 + j, N_DEV)
        my_out_rows = pl.ds(pos * M, M)

        @pl.when((j == 0) & (k == 0))
        def _():
            barrier_sem = pltpu.get_barrier_semaphore()
            for nbr in range(N_DEV):
                @pl.when(nbr != pos)
                def _(nbr=nbr):
                    pl.semaphore_signal(
                        barrier_sem, inc=1,
                        device_id=(nbr,), device_id_type=pl.DeviceIdType.MESH,
                    )
            pl.semaphore_wait(barrier_sem, N_DEV - 1)

        partial = jnp.dot(
            x_ref[...].astype(jnp.bfloat16),
            w_ref[...].astype(jnp.bfloat16),
            preferred_element_type=jnp.float32,
        )

        @pl.when(k == 0)
        def _():
            acc_ref[...] = partial

        @pl.when(k != 0)
        def _():
            acc_ref[...] += partial

        @pl.when(k == NK - 1)
        def _():
            @pl.when(j < N_DEV - 1)
            def _():
                rows = pl.ds(j * M, M)
                sendb_ref[rows, :] = jnp.maximum(acc_ref[...], 0.0).astype(
                    jnp.bfloat16
                )
                rdma = pltpu.make_async_remote_copy(
                    src_ref=sendb_ref.at[rows, :],
                    dst_ref=stage_ref.at[rows, :],
                    send_sem=send_sems.at[j],
                    recv_sem=recv_sems.at[j],
                    device_id=(dest,),
                    device_id_type=pl.DeviceIdType.MESH,
                )
                if COMM:
                    rdma.start()

            @pl.when(j == N_DEV - 1)
            def _():
                acc_ref[...] = jnp.maximum(acc_ref[...], 0.0)
                own = pltpu.make_async_copy(
                    acc_ref, out_ref.at[my_out_rows, :], copy_sem
                )
                own.start()
                own.wait()

                for r in range(N_DEV - 1):
                    src = lax.rem(pos - 1 - r + N_DEV, N_DEV)
                    rrows = pl.ds(r * M, M)
                    recv = pltpu.make_async_remote_copy(
                        src_ref=sendb_ref.at[rrows, :],
                        dst_ref=stage_ref.at[rrows, :],
                        send_sem=send_sems.at[r],
                        recv_sem=recv_sems.at[r],
                        device_id=(pos,),
                        device_id_type=pl.DeviceIdType.MESH,
                    )
                    if COMM:
                        recv.wait_recv()
                    acc_ref[...] = stage_ref[rrows, :].astype(jnp.float32)
                    store = pltpu.make_async_copy(
                        acc_ref, out_ref.at[pl.ds(src * M, M), :], copy_sem
                    )
                    store.start()
                    store.wait()

                for r in range(N_DEV - 1) if COMM else []:
                    send = pltpu.make_async_remote_copy(
                        src_ref=sendb_ref.at[pl.ds(r * M, M), :],
                        dst_ref=stage_ref.at[pl.ds(r * M, M), :],
                        send_sem=send_sems.at[r],
                        recv_sem=recv_sems.at[r],
                        device_id=(pos,),
                        device_id_type=pl.DeviceIdType.MESH,
                    )
                    send.wait_send()

    grid_spec = pltpu.PrefetchScalarGridSpec(
        num_scalar_prefetch=1,
        grid=(N_DEV, NK),
        in_specs=[
            pl.BlockSpec((M, KT), lambda j, k, pos_ref: (0, k)),
            pl.BlockSpec(
                (KT, NB),
                lambda j, k, pos_ref: (k, lax.rem(pos_ref[0] + 1 + j, N_DEV)),
            ),
        ],
        out_specs=pl.BlockSpec(memory_space=pl.ANY),
        scratch_shapes=[
            pltpu.VMEM((M, NB), jnp.float32),
            pltpu.VMEM((3 * M, NB), jnp.bfloat16),
            pltpu.VMEM((3 * M, NB), jnp.bfloat16),
            pltpu.SemaphoreType.DMA((N_DEV - 1,)),
            pltpu.SemaphoreType.DMA((N_DEV - 1,)),
            pltpu.SemaphoreType.DMA,
        ],
    )

    pos_arr = jnp.full((1,), lax.axis_index("i"), dtype=jnp.int32)
    return pl.pallas_call(
        body,
        grid_spec=grid_spec,
        out_shape=jax.ShapeDtypeStruct((N_DEV * M, NB), jnp.float32),
        compiler_params=pltpu.CompilerParams(
            collective_id=0,
            dimension_semantics=("arbitrary", "arbitrary"),
            vmem_limit_bytes=64 * 1024 * 1024,
        ),
    )(pos_arr, x, w_mat)


# device time: 256614 ns/iter; 1.0141x vs baseline; 1.0141x over previous
import jax
import jax.numpy as jnp
from jax import lax
from jax.experimental import pallas as pl
from jax.experimental.pallas import tpu as pltpu

N_DEV = 4
KT = 512
COMM = True


def kernel(x, w_mat):
    M, K = x.shape
    _, N = w_mat.shape
    NB = N // N_DEV
    NK = K // KT

    def body(pos_ref, x_ref, w_ref, out_ref, acc_ref, sendb_ref, stage_ref,
             send_sems, recv_sems, copy_sem):
        j = pl.program_id(0)
        k = pl.program_id(1)
        pos = pos_ref[0]
        dest = lax.rem(pos + 1 + j, N_DEV)
        my_out_rows = pl.ds(pos * M, M)

        @pl.when((j == 0) & (k == 0))
        def _():
            barrier_sem = pltpu.get_barrier_semaphore()
            for nbr in range(N_DEV):
                @pl.when(nbr != pos)
                def _(nbr=nbr):
                    pl.semaphore_signal(
                        barrier_sem, inc=1,
                        device_id=(nbr,), device_id_type=pl.DeviceIdType.MESH,
                    )
            pl.semaphore_wait(barrier_sem, N_DEV - 1)

        xb = x_ref[...]
        ph = NB // 2
        pa = jnp.dot(
            xb, w_ref[:, :ph], preferred_element_type=jnp.float32
        )
        pb = jnp.dot(
            xb, w_ref[:, ph:], preferred_element_type=jnp.float32
        )

        @pl.when(k == 0)
        def _():
            acc_ref[:, :ph] = pa
            acc_ref[:, ph:] = pb

        @pl.when(k != 0)
        def _():
            acc_ref[:, :ph] += pa
            acc_ref[:, ph:] += pb

        @pl.when(k == NK - 1)
        def _():
            @pl.when(j < N_DEV - 1)
            def _():
                rows = pl.ds(j * M, M)
                sendb_ref[rows, :] = jnp.maximum(acc_ref[...], 0.0).astype(
                    jnp.bfloat16
                )
                rdma = pltpu.make_async_remote_copy(
                    src_ref=sendb_ref.at[rows, :],
                    dst_ref=stage_ref.at[rows, :],
                    send_sem=send_sems.at[j],
                    recv_sem=recv_sems.at[j],
                    device_id=(dest,),
                    device_id_type=pl.DeviceIdType.MESH,
                )
                if COMM:
                    rdma.start()

            @pl.when(j == N_DEV - 1)
            def _():
                acc_ref[...] = jnp.maximum(acc_ref[...], 0.0)
                own = pltpu.make_async_copy(
                    acc_ref, out_ref.at[my_out_rows, :], copy_sem
                )
                own.start()
                own.wait()

                for r in range(N_DEV - 1):
                    src = lax.rem(pos - 1 - r + N_DEV, N_DEV)
                    rrows = pl.ds(r * M, M)
                    recv = pltpu.make_async_remote_copy(
                        src_ref=sendb_ref.at[rrows, :],
                        dst_ref=stage_ref.at[rrows, :],
                        send_sem=send_sems.at[r],
                        recv_sem=recv_sems.at[r],
                        device_id=(pos,),
                        device_id_type=pl.DeviceIdType.MESH,
                    )
                    if COMM:
                        recv.wait_recv()
                    acc_ref[...] = stage_ref[rrows, :].astype(jnp.float32)
                    store = pltpu.make_async_copy(
                        acc_ref, out_ref.at[pl.ds(src * M, M), :], copy_sem
                    )
                    store.start()
                    store.wait()

                for r in range(N_DEV - 1) if COMM else []:
                    send = pltpu.make_async_remote_copy(
                        src_ref=sendb_ref.at[pl.ds(r * M, M), :],
                        dst_ref=stage_ref.at[pl.ds(r * M, M), :],
                        send_sem=send_sems.at[r],
                        recv_sem=recv_sems.at[r],
                        device_id=(pos,),
                        device_id_type=pl.DeviceIdType.MESH,
                    )
                    send.wait_send()

    grid_spec = pltpu.PrefetchScalarGridSpec(
        num_scalar_prefetch=1,
        grid=(N_DEV, NK),
        in_specs=[
            pl.BlockSpec((M, KT), lambda j, k, pos_ref: (0, k)),
            pl.BlockSpec(
                (KT, NB),
                lambda j, k, pos_ref: (k, lax.rem(pos_ref[0] + 1 + j, N_DEV)),
            ),
        ],
        out_specs=pl.BlockSpec(memory_space=pl.ANY),
        scratch_shapes=[
            pltpu.VMEM((M, NB), jnp.float32),
            pltpu.VMEM((3 * M, NB), jnp.bfloat16),
            pltpu.VMEM((3 * M, NB), jnp.bfloat16),
            pltpu.SemaphoreType.DMA((N_DEV - 1,)),
            pltpu.SemaphoreType.DMA((N_DEV - 1,)),
            pltpu.SemaphoreType.DMA,
        ],
    )

    pos_arr = jnp.full((1,), lax.axis_index("i"), dtype=jnp.int32)
    return pl.pallas_call(
        body,
        grid_spec=grid_spec,
        out_shape=jax.ShapeDtypeStruct((N_DEV * M, NB), jnp.float32),
        compiler_params=pltpu.CompilerParams(
            collective_id=0,
            dimension_semantics=("arbitrary", "arbitrary"),
            vmem_limit_bytes=64 * 1024 * 1024,
        ),
    )(pos_arr, x, w_mat)


# device time: 223851 ns/iter; 1.1625x vs baseline; 1.1464x over previous
import jax
import jax.numpy as jnp
from jax import lax
from jax.experimental import pallas as pl
from jax.experimental.pallas import tpu as pltpu

N_DEV = 4
KT = 1024


def kernel(x, w_mat):
    M, K = x.shape
    _, N = w_mat.shape
    NB = N // N_DEV
    NK = K // KT

    def body(pos_ref, x_ref, w_ref, out_ref, acc_ref, sendb_ref, stage_ref,
             send_sems, recv_sems, copy_sem):
        j = pl.program_id(0)
        k = pl.program_id(1)
        pos = pos_ref[0]
        dest = lax.rem(pos + 1 + j, N_DEV)
        my_out_rows = pl.ds(pos * M, M)

        @pl.when((j == 0) & (k == 0))
        def _():
            barrier_sem = pltpu.get_barrier_semaphore()
            for nbr in range(N_DEV):
                @pl.when(nbr != pos)
                def _(nbr=nbr):
                    pl.semaphore_signal(
                        barrier_sem, inc=1,
                        device_id=(nbr,), device_id_type=pl.DeviceIdType.MESH,
                    )
            pl.semaphore_wait(barrier_sem, N_DEV - 1)

        partial = jnp.dot(
            x_ref[...], w_ref[...], preferred_element_type=jnp.float32
        )

        @pl.when(k == 0)
        def _():
            acc_ref[...] = partial

        @pl.when(k != 0)
        def _():
            acc_ref[...] += partial

        @pl.when(k == NK - 1)
        def _():
            @pl.when(j < N_DEV - 1)
            def _():
                slot = lax.rem(j, 2)
                srows = pl.ds(slot * M, M)

                @pl.when(j == 2)
                def _():
                    prev = pltpu.make_async_remote_copy(
                        src_ref=sendb_ref.at[pl.ds(0, M), :],
                        dst_ref=stage_ref.at[pl.ds(0, M), :],
                        send_sem=send_sems.at[0],
                        recv_sem=recv_sems.at[0],
                        device_id=(pos,),
                        device_id_type=pl.DeviceIdType.MESH,
                    )
                    prev.wait_send()

                sendb_ref[srows, :] = jnp.maximum(acc_ref[...], 0.0).astype(
                    jnp.bfloat16
                )
                rdma = pltpu.make_async_remote_copy(
                    src_ref=sendb_ref.at[srows, :],
                    dst_ref=stage_ref.at[pl.ds(j * M, M), :],
                    send_sem=send_sems.at[slot],
                    recv_sem=recv_sems.at[j],
                    device_id=(dest,),
                    device_id_type=pl.DeviceIdType.MESH,
                )
                rdma.start()

            @pl.when(j == N_DEV - 1)
            def _():
                acc_ref[...] = jnp.maximum(acc_ref[...], 0.0)
                own = pltpu.make_async_copy(
                    acc_ref, out_ref.at[my_out_rows, :], copy_sem
                )
                own.start()
                own.wait()

                for r in range(N_DEV - 1):
                    src = lax.rem(pos - 1 - r + N_DEV, N_DEV)
                    rrows = pl.ds(r * M, M)
                    recv = pltpu.make_async_remote_copy(
                        src_ref=sendb_ref.at[pl.ds(0, M), :],
                        dst_ref=stage_ref.at[rrows, :],
                        send_sem=send_sems.at[0],
                        recv_sem=recv_sems.at[r],
                        device_id=(pos,),
                        device_id_type=pl.DeviceIdType.MESH,
                    )
                    recv.wait_recv()
                    acc_ref[...] = stage_ref[rrows, :].astype(jnp.float32)
                    store = pltpu.make_async_copy(
                        acc_ref, out_ref.at[pl.ds(src * M, M), :], copy_sem
                    )
                    store.start()
                    store.wait()

                for slot in (1, 0):
                    send = pltpu.make_async_remote_copy(
                        src_ref=sendb_ref.at[pl.ds(slot * M, M), :],
                        dst_ref=stage_ref.at[pl.ds(slot * M, M), :],
                        send_sem=send_sems.at[slot],
                        recv_sem=recv_sems.at[0],
                        device_id=(pos,),
                        device_id_type=pl.DeviceIdType.MESH,
                    )
                    send.wait_send()

    grid_spec = pltpu.PrefetchScalarGridSpec(
        num_scalar_prefetch=1,
        grid=(N_DEV, NK),
        in_specs=[
            pl.BlockSpec((M, KT), lambda j, k, pos_ref: (0, k)),
            pl.BlockSpec(
                (KT, NB),
                lambda j, k, pos_ref: (k, lax.rem(pos_ref[0] + 1 + j, N_DEV)),
            ),
        ],
        out_specs=pl.BlockSpec(memory_space=pl.ANY),
        scratch_shapes=[
            pltpu.VMEM((M, NB), jnp.float32),
            pltpu.VMEM((2 * M, NB), jnp.bfloat16),
            pltpu.VMEM((3 * M, NB), jnp.bfloat16),
            pltpu.SemaphoreType.DMA((2,)),
            pltpu.SemaphoreType.DMA((N_DEV - 1,)),
            pltpu.SemaphoreType.DMA,
        ],
    )

    pos_arr = jnp.full((1,), lax.axis_index("i"), dtype=jnp.int32)
    return pl.pallas_call(
        body,
        grid_spec=grid_spec,
        out_shape=jax.ShapeDtypeStruct((N_DEV * M, NB), jnp.float32),
        compiler_params=pltpu.CompilerParams(
            collective_id=0,
            dimension_semantics=("arbitrary", "arbitrary"),
            vmem_limit_bytes=64 * 1024 * 1024,
        ),
    )(pos_arr, x, w_mat)
